# baseline (speedup 1.0000x reference)
import functools

import jax
import jax.numpy as jnp
from jax import lax
from jax.experimental import pallas as pl
from jax.experimental.pallas import tpu as pltpu
from jax.experimental.pallas import tpu_sc as plsc

_NC = 2
_NS = 16
_NW = _NC * _NS


def _gather_body(x_hbm, table_hbm, out_hbm, idx_v, rows_v, sem, *, per_w, chunk):
    wid = lax.axis_index("s") * _NC + lax.axis_index("c")
    base = wid * per_w
    n_chunks = per_w // chunk

    @pl.loop(0, n_chunks)
    def _(i):
        off = base + i * chunk
        pltpu.sync_copy(x_hbm.at[pl.ds(off, chunk)], idx_v)
        pltpu.async_copy(table_hbm.at[idx_v], rows_v, sem).wait()
        pltpu.sync_copy(rows_v, out_hbm.at[pl.ds(off, chunk)])


def kernel(x, table):
    b, s = x.shape
    n, d = table.shape
    total = b * s
    per_w = total // _NW
    chunk = 128
    x_flat = x.reshape(total).astype(jnp.int32)

    mesh = plsc.VectorSubcoreMesh(core_axis_name="c", subcore_axis_name="s")
    k = pl.kernel(
        functools.partial(_gather_body, per_w=per_w, chunk=chunk),
        out_type=jax.ShapeDtypeStruct((total, d), jnp.float32),
        mesh=mesh,
        compiler_params=pltpu.CompilerParams(use_tc_tiling_on_sc=False),
        scratch_types=[
            pltpu.VMEM((chunk,), jnp.int32),
            pltpu.VMEM((chunk, d), jnp.float32),
            pltpu.SemaphoreType.DMA,
        ],
    )
    out = k(x_flat, table)
    return out.reshape(b, s, d)


# speedup vs baseline: 1.1901x; 1.1901x over previous
import functools

import jax
import jax.numpy as jnp
from jax import lax
from jax.experimental import pallas as pl
from jax.experimental.pallas import tpu as pltpu
from jax.experimental.pallas import tpu_sc as plsc

_NC = 2
_NS = 16
_NW = _NC * _NS


def _gather_body(x_hbm, table_hbm, out_hbm, idx_all, rows, *sems,
                 per_w, chunk, nbuf):
    gsems, wsems = sems[:nbuf], sems[nbuf:]
    wid = lax.axis_index("s") * _NC + lax.axis_index("c")
    base = wid * per_w
    n_chunks = per_w // chunk

    pltpu.sync_copy(x_hbm.at[wid], idx_all)

    def gather(i, b):
        return pltpu.make_async_copy(
            table_hbm.at[idx_all.at[i]], rows.at[b], gsems[b])

    def writeback(i, b):
        return pltpu.make_async_copy(
            rows.at[b], out_hbm.at[pl.ds(base + i * chunk, chunk)], wsems[b])

    for b in range(nbuf):
        gather(b, b).start()

    @pl.loop(0, n_chunks, step=nbuf)
    def _(c0):
        for b in range(nbuf):
            i = c0 + b
            gather(i, b).wait()
            writeback(i, b).start()
        for b in range(nbuf):
            i = c0 + b
            writeback(i, b).wait()

            @pl.when(i + nbuf < n_chunks)
            def _():
                gather(i + nbuf, b).start()


def kernel(x, table):
    b, s = x.shape
    n, d = table.shape
    total = b * s
    per_w = total // _NW
    chunk = 256
    nbuf = 4
    n_chunks = per_w // chunk
    x_split = x.reshape(_NW, n_chunks, chunk).astype(jnp.int32)

    mesh = plsc.VectorSubcoreMesh(core_axis_name="c", subcore_axis_name="s")
    k = pl.kernel(
        functools.partial(_gather_body, per_w=per_w, chunk=chunk, nbuf=nbuf),
        out_type=jax.ShapeDtypeStruct((total, d), jnp.float32),
        mesh=mesh,
        compiler_params=pltpu.CompilerParams(use_tc_tiling_on_sc=False),
        scratch_types=(
            [pltpu.VMEM((n_chunks, chunk), jnp.int32),
             pltpu.VMEM((nbuf, chunk, d), jnp.float32)]
            + [pltpu.SemaphoreType.DMA] * (2 * nbuf)
        ),
    )
    out = k(x_split, table)
    return out.reshape(b, s, d)


# speedup vs baseline: 1.4413x; 1.2110x over previous
import functools

import jax
import jax.numpy as jnp
from jax import lax
from jax.experimental import pallas as pl
from jax.experimental.pallas import tpu as pltpu
from jax.experimental.pallas import tpu_sc as plsc

_NC = 2
_NS = 16
_NW = _NC * _NS


def _gather_body(x_hbm, table_hbm, out_hbm, idx_all, rows, *sems,
                 per_w, chunk, nbuf):
    gsems, wsems = sems[:nbuf], sems[nbuf:]
    wid = lax.axis_index("s") * _NC + lax.axis_index("c")
    base = wid * per_w
    n_chunks = per_w // chunk

    pltpu.sync_copy(x_hbm.at[wid], idx_all)

    def gather(i, b):
        return pltpu.make_async_copy(
            table_hbm.at[idx_all.at[i]], rows.at[b], gsems[b])

    def writeback(i, b):
        return pltpu.make_async_copy(
            rows.at[b], out_hbm.at[pl.ds(base + i * chunk, chunk)], wsems[b])

    for b in range(nbuf):
        gather(b, b).start()

    @pl.loop(0, n_chunks, step=nbuf)
    def _(c0):
        for b in range(nbuf):
            i = c0 + b
            gather(i, b).wait()
            writeback(i, b).start()
        for b in range(nbuf):
            i = c0 + b
            writeback(i, b).wait()

            @pl.when(i + nbuf < n_chunks)
            def _():
                gather(i + nbuf, b).start()


def kernel(x, table):
    b, s = x.shape
    n, d = table.shape
    dp = 128
    total = b * s
    per_w = total // _NW
    chunk = 256
    nbuf = 2
    n_chunks = per_w // chunk
    x_split = x.reshape(_NW, n_chunks, chunk).astype(jnp.int32)
    tpad = jnp.pad(table, ((0, 0), (0, dp - d)))

    mesh = plsc.VectorSubcoreMesh(core_axis_name="c", subcore_axis_name="s")
    k = pl.kernel(
        functools.partial(_gather_body, per_w=per_w, chunk=chunk, nbuf=nbuf),
        out_type=jax.ShapeDtypeStruct((total, dp), jnp.float32),
        mesh=mesh,
        compiler_params=pltpu.CompilerParams(use_tc_tiling_on_sc=False),
        scratch_types=(
            [pltpu.VMEM((n_chunks, chunk), jnp.int32),
             pltpu.VMEM((nbuf, chunk, dp), jnp.float32)]
            + [pltpu.SemaphoreType.DMA] * (2 * nbuf)
        ),
    )
    out = k(x_split, tpad)
    return out.reshape(b, s, dp)[:, :, :d]
